# Initial kernel scaffold; baseline (speedup 1.0000x reference)
import jax
import jax.numpy as jnp
from jax.experimental import pallas as pl


def kernel(input_feature, coords, edge_index, edge_index_2rd, edx_jk, edx_ij, batch, num_edge_inside, edge_rep, geo_W0, geo_b0, geo_W1, geo_b1, att, Wfirst, bfirst, Wrest, brest):
    raise NotImplementedError("write your pallas kernel here")


# speedup vs baseline: 2.3384x; 2.3384x over previous
import functools

import jax
import jax.numpy as jnp
from jax import lax
from jax.experimental import pallas as pl
from jax.experimental.pallas import tpu as pltpu
from jax.experimental.pallas import tpu_sc as plsc

_NW = 32
def _sc_gather(table, idx):
    b_total = idx.shape[0]
    per_w = b_total // _NW
    assert per_w * _NW == b_total and per_w % 8 == 0
    chunk = 5000
    assert per_w % chunk == 0
    n_chunks = per_w // chunk
    mesh = plsc.VectorSubcoreMesh(core_axis_name="c", subcore_axis_name="s")

    @functools.partial(
        pl.kernel,
        out_type=jax.ShapeDtypeStruct((b_total, 16), jnp.float32),
        mesh=mesh,
        compiler_params=pltpu.CompilerParams(use_tc_tiling_on_sc=False),
        scratch_types=[
            pltpu.VMEM((chunk,), jnp.int32),
            pltpu.VMEM((chunk, 16), jnp.float32),
            pltpu.SemaphoreType.DMA,
        ],
    )
    def gather_kernel(table_hbm, idx_hbm, out_hbm, idx_v, rows_v, sem):
        wid = lax.axis_index("s") * 2 + lax.axis_index("c")
        base = wid * per_w

        def body(g, carry):
            off = base + g * chunk
            pltpu.sync_copy(idx_hbm.at[pl.ds(off, chunk)], idx_v)
            pltpu.async_copy(table_hbm.at[idx_v], rows_v, sem).wait()
            pltpu.sync_copy(rows_v, out_hbm.at[pl.ds(off, chunk)])
            return carry

        lax.fori_loop(0, n_chunks, body, 0)

    return gather_kernel(table, idx)


def _sc_segsum(x, idx2d, zeros_np):
    t_rows = x.shape[0]
    ir = t_rows // 128
    np_rows = zeros_np.shape[0]
    per_tile_np = np_rows // 16
    assert per_tile_np % 8 == 0
    ch = 8
    full = ir // ch
    rem = ir - full * ch
    n_iter = (full + 1 + _NW - 1) // _NW
    mesh = plsc.VectorSubcoreMesh(core_axis_name="c", subcore_axis_name="s")

    @functools.partial(
        pl.kernel,
        out_type=jax.ShapeDtypeStruct((2, np_rows, 16), jnp.float32),
        mesh=mesh,
        compiler_params=pltpu.CompilerParams(use_tc_tiling_on_sc=False),
        scratch_types=[
            pltpu.VMEM((ch, 128), jnp.int32),
            pltpu.VMEM((ch * 128, 16), jnp.float32),
            pltpu.VMEM_SHARED((np_rows, 16), jnp.float32),
        ],
    )
    def segsum_kernel(x_hbm, idx_hbm, zeros_hbm, out_hbm, idx_v, rows_v, acc):
        c = lax.axis_index("c")
        s = lax.axis_index("s")
        w = s * 2 + c
        pltpu.sync_copy(zeros_hbm.at[pl.ds(s * per_tile_np, per_tile_np)],
                        acc.at[pl.ds(s * per_tile_np, per_tile_np)])
        plsc.subcore_barrier()

        def body(g, carry):
            q = w + g * _NW

            @pl.when(q < full)
            def _():
                pltpu.sync_copy(idx_hbm.at[pl.ds(q * ch, ch)], idx_v)
                pltpu.sync_copy(x_hbm.at[pl.ds(q * ch * 128, ch * 128)], rows_v)
                for kk in range(ch):
                    pltpu.sync_copy(rows_v.at[pl.ds(kk * 128, 128)],
                                    acc.at[idx_v.at[kk]], add=True)

            if rem:
                @pl.when(q == full)
                def _():
                    pltpu.sync_copy(idx_hbm.at[pl.ds(full * ch, rem)],
                                    idx_v.at[pl.ds(0, rem)])
                    pltpu.sync_copy(x_hbm.at[pl.ds(full * ch * 128, rem * 128)],
                                    rows_v.at[pl.ds(0, rem * 128)])
                    for kk in range(rem):
                        pltpu.sync_copy(rows_v.at[pl.ds(kk * 128, 128)],
                                        acc.at[idx_v.at[kk]], add=True)
            return carry

        lax.fori_loop(0, n_iter, body, 0)
        plsc.subcore_barrier()
        pltpu.sync_copy(acc.at[pl.ds(s * per_tile_np, per_tile_np)],
                        out_hbm.at[c, pl.ds(s * per_tile_np, per_tile_np)])

    return segsum_kernel(x, idx2d, zeros_np)


_MM = (((1,), (0,)), ((), ()))


def _dot(a, b):
    return lax.dot_general(a, b, _MM, preferred_element_type=jnp.float32,
                           precision=lax.Precision.HIGHEST)


def _rsqrt(x):
    r = lax.rsqrt(x)
    return r * (1.5 - 0.5 * x * r * r)


def _dotw(a, b):
    return lax.dot_general(a, b, _MM, preferred_element_type=jnp.float32)


def _rowspec(br):
    return pl.BlockSpec((br, 128), lambda i: (i, 0))


def _fullspec(shape):
    return pl.BlockSpec(shape, lambda i: tuple(0 for _ in shape))


def _statspec():
    return pl.BlockSpec((8, 128), lambda i: (0, 0))


def _smemspec():
    return pl.BlockSpec(memory_space=pltpu.SMEM)


def _geo1_body(ci_ref, cj_ref, ck_ref, blk_ref, w0p_ref, b0p_ref,
               z_ref, st_ref):
    @pl.when(pl.program_id(0) == 0)
    def _():
        st_ref[...] = jnp.zeros_like(st_ref)

    v1 = cj_ref[...] - ci_ref[...]
    v2 = ck_ref[...] - cj_ref[...]
    blk = blk_ref[...]
    d1sq = _dot(v1 * v1, blk)
    d2sq = _dot(v2 * v2, blk)
    dot12 = _dot(v1 * v2, blk)
    d_ij = jnp.sqrt(d1sq)
    d_jk = jnp.sqrt(d2sq)
    cn = jnp.sqrt(jnp.maximum(d1sq * d2sq - dot12 * dot12, 0.0))
    theta = jnp.arctan2(cn, dot12)
    li = lax.broadcasted_iota(jnp.int32, (1, 128), 1) % 16
    g3 = (d_ij * (li == 0).astype(jnp.float32)
          + d_jk * (li == 1).astype(jnp.float32)
          + theta * (li == 2).astype(jnp.float32))
    z = _dotw(g3, w0p_ref[...]) + b0p_ref[...]
    z_ref[...] = z
    s = jnp.sum(z, axis=0, keepdims=True)
    q = jnp.sum(z * z, axis=0, keepdims=True)
    st_ref[...] = st_ref[...] + jnp.concatenate(
        [s, q, jnp.zeros((6, 128), jnp.float32)], axis=0)


def _geo2_body(n_rows, zp_ref, stp_ref, tile_ref, w1big_ref, b1p_ref,
               z_ref, st_ref):
    @pl.when(pl.program_id(0) == 0)
    def _():
        st_ref[...] = jnp.zeros_like(st_ref)

    stot = _dot(stp_ref[...], tile_ref[...])
    m = stot[0:1, :] / n_rows
    msq = stot[1:2, :] / n_rows
    inv = _rsqrt(msq - m * m + 1e-5)
    a = jnp.maximum((zp_ref[...] - m) * inv, 0.0)
    z = _dotw(a, w1big_ref[...]) + b1p_ref[...]
    z_ref[...] = z
    s = jnp.sum(z, axis=0, keepdims=True)
    q = jnp.sum(z * z, axis=0, keepdims=True)
    st_ref[...] = st_ref[...] + jnp.concatenate(
        [s, q, jnp.zeros((6, 128), jnp.float32)], axis=0)


def _geo3_body(n_rows, zp_ref, stp_ref, tile_ref, geo_ref):
    stot = _dot(stp_ref[...], tile_ref[...])
    m = stot[0:1, :] / n_rows
    msq = stot[1:2, :] / n_rows
    inv = _rsqrt(msq - m * m + 1e-5)
    geo_ref[...] = jnp.maximum((zp_ref[...] - m) * inv, 0.0)


def _first_body(thr_ref, nfi_ref, nfj_ref, nfk_ref, geo_ref, exi_ref, exj_ref,
                wfbig_ref, bfp_ref, z_ref, st_ref, ct_ref, bc_ref):
    @pl.when(pl.program_id(0) == 0)
    def _():
        st_ref[...] = jnp.zeros_like(st_ref)
        ct_ref[...] = jnp.zeros_like(ct_ref)

    thr = thr_ref[0]
    mi = exi_ref[...] < thr
    mj = exj_ref[...] < thr
    ni = jnp.logical_not(mi)
    nj = jnp.logical_not(mj)
    masks = [jnp.logical_and(mi, mj), jnp.logical_and(mi, nj),
             jnp.logical_and(ni, mj), jnp.logical_and(ni, nj)]
    parts = (nfi_ref[...], nfj_ref[...], nfk_ref[...], geo_ref[...])
    z = jnp.zeros_like(parts[0])
    bc = jnp.zeros_like(parts[0])
    sums, sqs, cts = [], [], []
    for b in range(4):
        zb = bfp_ref[b].reshape(1, 128)
        for p in range(4):
            zb = zb + _dotw(parts[p], wfbig_ref[b, p])
        mf = masks[b].astype(jnp.float32)
        zbm = mf * zb
        z = z + zbm
        bc = bc + mf * float(b)
        sums.append(jnp.sum(zbm, axis=0, keepdims=True))
        sqs.append(jnp.sum(zbm * zb, axis=0, keepdims=True))
        cts.append(jnp.full((1, 128), jnp.sum(mf) * (1.0 / 16.0), jnp.float32))
    z_ref[...] = z
    bc_ref[...] = bc
    st_ref[...] = st_ref[...] + jnp.concatenate(sums + sqs, axis=0)
    ct_ref[...] = ct_ref[...] + jnp.concatenate(
        cts + [jnp.zeros((4, 128), jnp.float32)], axis=0)


def _rest_body(zp_ref, bc_ref, stp_ref, ctp_ref, tile_ref, wrbig_ref, brp_ref,
               z_ref, st_ref):
    @pl.when(pl.program_id(0) == 0)
    def _():
        st_ref[...] = jnp.zeros_like(st_ref)

    bc = bc_ref[...]
    zprev = zp_ref[...]
    stot = _dot(stp_ref[...], tile_ref[...])
    ctp = ctp_ref[...]
    m_sel = jnp.zeros_like(zprev)
    inv_sel = jnp.zeros_like(zprev)
    mfs = []
    for b in range(4):
        mf = (bc == float(b)).astype(jnp.float32)
        mfs.append(mf)
        cnt = jnp.maximum(ctp[b:b + 1, :], 1.0)
        m = stot[b:b + 1, :] / cnt
        msq = stot[b + 4:b + 5, :] / cnt
        inv = _rsqrt(msq - m * m + 1e-5)
        m_sel = m_sel + mf * m
        inv_sel = inv_sel + mf * inv
    a = jnp.maximum((zprev - m_sel) * inv_sel, 0.0)
    z = jnp.zeros_like(zprev)
    sums, sqs = [], []
    for b in range(4):
        zb = _dotw(a, wrbig_ref[b]) + brp_ref[b].reshape(1, 128)
        zbm = mfs[b] * zb
        z = z + zbm
        sums.append(jnp.sum(zbm, axis=0, keepdims=True))
        sqs.append(jnp.sum(zbm * zb, axis=0, keepdims=True))
    z_ref[...] = z
    st_ref[...] = st_ref[...] + jnp.concatenate(sums + sqs, axis=0)


def _final_body(zp_ref, bc_ref, stp_ref, ctp_ref, tile_ref, attp_ref, x_ref):
    bc = bc_ref[...]
    zprev = zp_ref[...]
    stot = _dot(stp_ref[...], tile_ref[...])
    ctp = ctp_ref[...]
    m_sel = jnp.zeros_like(zprev)
    inv_sel = jnp.zeros_like(zprev)
    att_sel = jnp.zeros_like(zprev)
    for b in range(4):
        mf = (bc == float(b)).astype(jnp.float32)
        cnt = jnp.maximum(ctp[b:b + 1, :], 1.0)
        m = stot[b:b + 1, :] / cnt
        msq = stot[b + 4:b + 5, :] / cnt
        inv = _rsqrt(msq - m * m + 1e-5)
        m_sel = m_sel + mf * m
        inv_sel = inv_sel + mf * inv
        att_sel = att_sel + mf * attp_ref[b:b + 1, :]
    x_ref[...] = jnp.maximum((zprev - m_sel) * inv_sel, 0.0) * att_sel


def _sum_body(n8, p_ref, o_ref):
    p = p_ref[...]
    o_ref[...] = p[0, :n8, :] + p[1, :n8, :]


def _bd8(w):
    return jnp.kron(jnp.eye(8, dtype=jnp.float32), w.T)


def kernel(input_feature, coords, edge_index, edge_index_2rd, edx_jk, edx_ij,
           batch, num_edge_inside, edge_rep, geo_W0, geo_b0, geo_W1, geo_b1,
           att, Wfirst, bfirst, Wrest, brest):
    t_rows = edge_index_2rd.shape[1]
    n_nodes = input_feature.shape[0]
    n_iters = Wfirst.shape[0]
    n_deep = Wrest.shape[2]
    assert t_rows % 128 == 0 and n_nodes % 8 == 0
    t8 = t_rows // 8
    br = next(x for x in (2000, 2048, 1024, 512, 256, t8) if t8 % x == 0)
    grid = (t8 // br,)
    n8 = n_nodes // 8
    np_rows = ((n_nodes + 127) // 128) * 128

    thr = jnp.asarray(num_edge_inside, jnp.int32).reshape(1)
    flat_idx = edge_index_2rd.reshape(-1)
    exi = jnp.broadcast_to(edx_ij.astype(jnp.int32).reshape(t_rows, 1),
                           (t_rows, 16)).reshape(t8, 128)
    exj = jnp.broadcast_to(edx_jk.astype(jnp.int32).reshape(t_rows, 1),
                           (t_rows, 16)).reshape(t8, 128)
    idx2d = edge_index_2rd[0].reshape(t_rows // 128, 128)
    zeros_np = jnp.zeros((np_rows, 16), jnp.float32)
    coords_pad = jnp.pad(coords.astype(jnp.float32), ((0, 0), (0, 13)))

    tile128 = jnp.tile(jnp.eye(16, dtype=jnp.float32), (8, 8))
    blk128 = jnp.kron(jnp.eye(8, dtype=jnp.float32),
                      jnp.ones((16, 16), jnp.float32))
    w0p = jnp.kron(jnp.eye(8, dtype=jnp.float32),
                   jnp.pad(geo_W0.T, ((0, 13), (0, 0))))
    b0p = jnp.tile(geo_b0, 8).reshape(1, 128)
    b1p = jnp.tile(geo_b1, 8).reshape(1, 128)
    w1big = _bd8(geo_W1)
    wfbig = jnp.stack([
        jnp.stack([
            jnp.stack([_bd8(Wfirst[t, b, :, p * 16:(p + 1) * 16])
                       for p in range(4)])
            for b in range(4)])
        for t in range(n_iters)])
    bfp = jnp.tile(bfirst, (1, 1, 8))
    wrbig = jnp.stack([
        jnp.stack([
            jnp.stack([_bd8(Wrest[t, b, d]) for b in range(4)])
            for d in range(n_deep)])
        for t in range(n_iters)])
    brp = jnp.tile(jnp.swapaxes(brest, 1, 2), (1, 1, 1, 8))
    attp = jnp.broadcast_to(att.reshape(4, 1), (4, 128))

    f32 = jnp.float32
    big = lambda: jax.ShapeDtypeStruct((t8, 128), f32)
    st_s = lambda: jax.ShapeDtypeStruct((8, 128), f32)

    crows = _sc_gather(coords_pad, flat_idx)
    ci = crows[:t_rows].reshape(t8, 128)
    cj = crows[t_rows:2 * t_rows].reshape(t8, 128)
    ck = crows[2 * t_rows:].reshape(t8, 128)

    zg1, st1 = pl.pallas_call(
        _geo1_body, grid=grid,
        in_specs=[_rowspec(br), _rowspec(br), _rowspec(br),
                  _fullspec((128, 128)), _fullspec((128, 128)),
                  _fullspec((1, 128))],
        out_specs=[_rowspec(br), _statspec()],
        out_shape=[big(), st_s()],
    )(ci, cj, ck, blk128, w0p, b0p)

    zg2, st2 = pl.pallas_call(
        functools.partial(_geo2_body, float(t_rows)), grid=grid,
        in_specs=[_rowspec(br), _statspec(), _fullspec((128, 128)),
                  _fullspec((128, 128)), _fullspec((1, 128))],
        out_specs=[_rowspec(br), _statspec()],
        out_shape=[big(), st_s()],
    )(zg1, st1, tile128, w1big, b1p)

    geo = pl.pallas_call(
        functools.partial(_geo3_body, float(t_rows)), grid=grid,
        in_specs=[_rowspec(br), _statspec(), _fullspec((128, 128))],
        out_specs=_rowspec(br),
        out_shape=big(),
    )(zg2, st2, tile128)


    nf = input_feature
    outs = []
    for t in range(n_iters):
        nrows = _sc_gather(nf, flat_idx)
        nfi = nrows[:t_rows].reshape(t8, 128)
        nfj = nrows[t_rows:2 * t_rows].reshape(t8, 128)
        nfk = nrows[2 * t_rows:].reshape(t8, 128)

        z, st, ct, bcode = pl.pallas_call(
            _first_body, grid=grid,
            in_specs=[_smemspec(), _rowspec(br), _rowspec(br), _rowspec(br),
                      _rowspec(br), _rowspec(br), _rowspec(br),
                      _fullspec((4, 4, 128, 128)), _fullspec((4, 128))],
            out_specs=[_rowspec(br), _statspec(), _statspec(), _rowspec(br)],
            out_shape=[big(), st_s(), st_s(), big()],
        )(thr, nfi, nfj, nfk, geo, exi, exj, wfbig[t], bfp[t])

        for d in range(n_deep):
            z, st = pl.pallas_call(
                _rest_body, grid=grid,
                in_specs=[_rowspec(br), _rowspec(br), _statspec(), _statspec(),
                          _fullspec((128, 128)), _fullspec((4, 128, 128)),
                          _fullspec((4, 128))],
                out_specs=[_rowspec(br), _statspec()],
                out_shape=[big(), st_s()],
            )(z, bcode, st, ct, tile128, wrbig[t, d], brp[t, d])

        x = pl.pallas_call(
            _final_body, grid=grid,
            in_specs=[_rowspec(br), _rowspec(br), _statspec(), _statspec(),
                      _fullspec((128, 128)), _fullspec((4, 128))],
            out_specs=_rowspec(br),
            out_shape=big(),
        )(z, bcode, st, ct, tile128, attp)

        partials = _sc_segsum(x.reshape(t_rows, 16), idx2d, zeros_np)

        nf = pl.pallas_call(
            functools.partial(_sum_body, n8), grid=(1,),
            in_specs=[pl.BlockSpec((2, np_rows // 8, 128),
                                   lambda i: (0, 0, 0))],
            out_specs=pl.BlockSpec((n8, 128), lambda i: (0, 0)),
            out_shape=jax.ShapeDtypeStruct((n8, 128), f32),
        )(partials.reshape(2, np_rows // 8, 128)).reshape(n_nodes, 16)
        outs.append(nf)

    return tuple(outs)
